# Initial kernel scaffold; baseline (speedup 1.0000x reference)
import jax
import jax.numpy as jnp
from jax.experimental import pallas as pl


def kernel(inputs):
    raise NotImplementedError("write your pallas kernel here")


# speedup vs baseline: 220.7121x; 220.7121x over previous
import jax
import jax.numpy as jnp
from jax.experimental import pallas as pl

_D = 2048
_K = 1638
_ROWS_PER_BLOCK = 256
_BISECT_ITERS = 30


def _body(x_ref, o_ref):
    x = x_ref[...]
    s = jnp.sum(x * x, axis=1, keepdims=True)
    boost = jnp.exp(_K / jnp.sqrt(s))
    lo = jnp.min(x, axis=1, keepdims=True)
    hi = jnp.max(x, axis=1, keepdims=True)
    kf = jnp.float32(_K)

    def step(_, c):
        lo, hi = c
        mid = (lo + hi) * 0.5
        cnt = jnp.sum(jnp.where(x >= mid, 1.0, 0.0), axis=1, keepdims=True)
        ge = cnt >= kf
        return jnp.where(ge, mid, lo), jnp.where(ge, hi, mid)

    lo, hi = jax.lax.fori_loop(0, _BISECT_ITERS, step, (lo, hi))
    o_ref[...] = jnp.where(x >= lo, x * boost, 0.0)


def kernel(inputs):
    n, d = inputs.shape
    assert d == _D
    rb = min(n, _ROWS_PER_BLOCK)
    return pl.pallas_call(
        _body,
        grid=(n // rb,),
        in_specs=[pl.BlockSpec((rb, d), lambda i: (i, 0))],
        out_specs=pl.BlockSpec((rb, d), lambda i: (i, 0)),
        out_shape=jax.ShapeDtypeStruct((n, d), inputs.dtype),
    )(inputs)


# speedup vs baseline: 320.7848x; 1.4534x over previous
import jax
import jax.numpy as jnp
from jax.experimental import pallas as pl

_D = 2048
_K = 1638
_ROWS_PER_BLOCK = 256
_BISECT_ITERS = 20


def _body(x_ref, o_ref):
    x = x_ref[...]
    s = jnp.sum(x * x, axis=1, keepdims=True)
    boost = jnp.exp(_K / jnp.sqrt(s))
    lo = jnp.min(x, axis=1, keepdims=True)
    hi = jnp.max(x, axis=1, keepdims=True)
    kf = jnp.float32(_K)

    def step(_, c):
        lo, hi = c
        mid = (lo + hi) * 0.5
        cnt = jnp.sum(jnp.where(x >= mid, 1.0, 0.0), axis=1, keepdims=True)
        ge = cnt >= kf
        return jnp.where(ge, mid, lo), jnp.where(ge, hi, mid)

    lo, hi = jax.lax.fori_loop(0, _BISECT_ITERS, step, (lo, hi))
    o_ref[...] = jnp.where(x >= lo, x * boost, 0.0)


def kernel(inputs):
    n, d = inputs.shape
    assert d == _D
    rb = min(n, _ROWS_PER_BLOCK)
    return pl.pallas_call(
        _body,
        grid=(n // rb,),
        in_specs=[pl.BlockSpec((rb, d), lambda i: (i, 0))],
        out_specs=pl.BlockSpec((rb, d), lambda i: (i, 0)),
        out_shape=jax.ShapeDtypeStruct((n, d), inputs.dtype),
    )(inputs)
